# baseline (speedup 1.0000x reference)
import jax
import jax.numpy as jnp
from jax import lax
from jax.experimental import pallas as pl
from jax.experimental.pallas import tpu as pltpu
from jax.experimental.pallas import tpu_sc as plsc

N = 10000
D = 128
E = 320000

NC = 2
NS = 16
K = 128
CPT = 160
NCH = NS * CPT
E_PAD = NCH * K
N_PAD = 10240
RPT = N_PAD // NS
WSB = 2 * N_PAD


def _sc_body(src_hbm, dst_hbm, ew_hbm, feat_hbm, zrow_hbm, zws_hbm,
             hf_out, ws_out,
             sb0, sb1, db0, db1, wb0, wb1, rows0, rows1, wv, di2,
             hf_sh, ws_sh, semg0, semg1, sems0, sems1, seme0, seme1):
    cid = lax.axis_index("c")
    sid = lax.axis_index("s")
    base = sid * CPT

    @pl.when(cid == 0)
    def _rows_core():
        pltpu.sync_copy(zrow_hbm, rows0)
        for k in range(RPT // K):
            pltpu.sync_copy(rows0, hf_sh.at[pl.ds(sid * RPT + k * K, K)])
        plsc.subcore_barrier()

        pltpu.async_copy(src_hbm.at[base], sb0, seme0)
        pltpu.async_copy(dst_hbm.at[base], db0, seme0)

        def chunk(j, sb, db, rows_b, semg, sb_n, db_n, seme_n, seme_b):
            pltpu.make_async_copy(src_hbm.at[base], sb, seme_b).wait()
            pltpu.make_async_copy(dst_hbm.at[base], db, seme_b).wait()
            cp = pltpu.async_copy(feat_hbm.at[sb.at[0]], rows_b, semg)
            pltpu.async_copy(src_hbm.at[base + j + 1], sb_n, seme_n)
            pltpu.async_copy(dst_hbm.at[base + j + 1], db_n, seme_n)
            cp.wait()
            pltpu.sync_copy(rows_b, hf_sh.at[db.at[0]], add=True)

        def body(i, carry):
            chunk(2 * i, sb0, db0, rows0, semg0, sb1, db1, seme1, seme0)
            chunk(2 * i + 1, sb1, db1, rows1, semg1, sb0, db0, seme0, seme1)
            return carry

        lax.fori_loop(0, CPT // 2, body, 0)
        pltpu.make_async_copy(src_hbm.at[base], sb0, seme0).wait()
        pltpu.make_async_copy(dst_hbm.at[base], db0, seme0).wait()
        plsc.subcore_barrier()

        pltpu.sync_copy(hf_sh.at[pl.ds(sid * RPT, RPT)],
                        hf_out.at[pl.ds(sid * RPT, RPT)])

    @pl.when(cid != 0)
    def _scalar_core():
        pltpu.sync_copy(zws_hbm, ws_sh.at[pl.ds(sid * 2 * RPT, 2 * RPT)])
        plsc.subcore_barrier()

        pltpu.async_copy(dst_hbm.at[base], db0, seme0)
        pltpu.async_copy(ew_hbm.at[base], wb0, seme0)

        def chunk(j, db, wb, db_n, wb_n, seme_n, seme_b):
            pltpu.make_async_copy(dst_hbm.at[base], db, seme_b).wait()
            pltpu.make_async_copy(ew_hbm.at[base], wb, seme_b).wait()
            pltpu.async_copy(dst_hbm.at[base + j + 1], db_n, seme_n)
            pltpu.async_copy(ew_hbm.at[base + j + 1], wb_n, seme_n)
            for v in range(K // 16):
                w = wb[0, pl.ds(v * 16, 16)]
                d = db[0, pl.ds(v * 16, 16)]
                fp = ((d >> 10) << 11) + (d & 1023)
                wv[0, pl.ds(v * 16, 16)] = jnp.maximum(w, 0.0)
                wv[1, pl.ds(v * 16, 16)] = jnp.maximum(-w, 0.0)
                di2[0, pl.ds(v * 16, 16)] = fp
                di2[1, pl.ds(v * 16, 16)] = fp + 1024
            pltpu.sync_copy(wv.at[0], ws_sh.at[di2.at[0]], add=True)
            pltpu.sync_copy(wv.at[1], ws_sh.at[di2.at[1]], add=True)

        def body(i, carry):
            chunk(2 * i, db0, wb0, db1, wb1, seme1, seme0)
            chunk(2 * i + 1, db1, wb1, db0, wb0, seme0, seme1)
            return carry

        lax.fori_loop(0, CPT // 2, body, 0)
        pltpu.make_async_copy(dst_hbm.at[base], db0, seme0).wait()
        pltpu.make_async_copy(ew_hbm.at[base], wb0, seme0).wait()
        plsc.subcore_barrier()

        pltpu.sync_copy(ws_sh.at[pl.ds(sid * 2 * RPT, 2 * RPT)],
                        ws_out.at[pl.ds(sid * 2 * RPT, 2 * RPT)])


def _sc_call(src2, dst2, ew2, feat, zrow, zws):
    mesh = plsc.VectorSubcoreMesh(core_axis_name="c", subcore_axis_name="s")
    f = pl.kernel(
        _sc_body,
        out_type=[
            jax.ShapeDtypeStruct((N_PAD, D), jnp.float32),
            jax.ShapeDtypeStruct((WSB,), jnp.float32),
        ],
        mesh=mesh,
        scratch_types=[
            pltpu.VMEM((1, K), jnp.int32),
            pltpu.VMEM((1, K), jnp.int32),
            pltpu.VMEM((1, K), jnp.int32),
            pltpu.VMEM((1, K), jnp.int32),
            pltpu.VMEM((1, K), jnp.float32),
            pltpu.VMEM((1, K), jnp.float32),
            pltpu.VMEM((K, D), jnp.float32),
            pltpu.VMEM((K, D), jnp.float32),
            pltpu.VMEM((2, K), jnp.float32),
            pltpu.VMEM((2, K), jnp.int32),
            pltpu.VMEM_SHARED((N_PAD, D), jnp.float32),
            pltpu.VMEM_SHARED((WSB,), jnp.float32),
            pltpu.SemaphoreType.DMA,
            pltpu.SemaphoreType.DMA,
            pltpu.SemaphoreType.DMA,
            pltpu.SemaphoreType.DMA,
            pltpu.SemaphoreType.DMA,
            pltpu.SemaphoreType.DMA,
        ],
    )
    return f(src2, dst2, ew2, feat, zrow, zws)


def _tc_epilogue(x_ref, hf_ref, ws_ref, wx_ref, wf_ref, ww_ref, b_ref, wt_ref,
                 out_ref):
    f32 = jnp.float32
    wt = wt_ref[...]
    rw = jnp.concatenate([jnp.maximum(wt, 0.0), jnp.maximum(-wt, 0.0)], axis=0)
    v = lax.dot_general(rw, ww_ref[...], (((1,), (1,)), ((), ())),
                        preferred_element_type=f32)
    s2 = ws_ref[0]
    acc = lax.dot_general(x_ref[...], wx_ref[...], (((1,), (1,)), ((), ())),
                          preferred_element_type=f32)
    acc += lax.dot_general(hf_ref[...], wf_ref[...], (((1,), (1,)), ((), ())),
                           preferred_element_type=f32)
    acc += lax.dot_general(s2, v, (((0,), (0,)), ((), ())),
                           preferred_element_type=f32)
    acc += b_ref[...]
    out_ref[...] = jnp.maximum(acc, 0.0)


def _tc_call(x, hf, ws, W_x, W_f, W_w, b_f, weights):
    blk = 1024
    grid = (N_PAD // blk,)
    return pl.pallas_call(
        _tc_epilogue,
        grid=grid,
        in_specs=[
            pl.BlockSpec((blk, D), lambda i: (i, 0)),
            pl.BlockSpec((blk, D), lambda i: (i, 0)),
            pl.BlockSpec((1, 2, blk), lambda i: (i, 0, 0)),
            pl.BlockSpec((D, D), lambda i: (0, 0)),
            pl.BlockSpec((D, D), lambda i: (0, 0)),
            pl.BlockSpec((D, D), lambda i: (0, 0)),
            pl.BlockSpec((1, D), lambda i: (0, 0)),
            pl.BlockSpec((1, D), lambda i: (0, 0)),
        ],
        out_specs=pl.BlockSpec((blk, D), lambda i: (i, 0)),
        out_shape=jax.ShapeDtypeStruct((N_PAD, D), jnp.float32),
    )(x, hf, ws, W_x, W_f, W_w, b_f, weights)


@jax.jit
def kernel(x, feat, edge_index, edge_w, W_x, W_w, W_f, b_f, weights):
    src = edge_index[0].astype(jnp.int32)
    dst = edge_index[1].astype(jnp.int32)
    pad = E_PAD - E
    pad_dst = N + jnp.arange(pad, dtype=jnp.int32) % (N_PAD - N)
    zk = jnp.zeros((K,), jnp.int32)
    src2 = jnp.concatenate([src, jnp.zeros((pad,), jnp.int32),
                            zk]).reshape(NCH + 1, 1, K)
    dst2 = jnp.concatenate([dst, pad_dst,
                            zk + N]).reshape(NCH + 1, 1, K)
    ew2 = jnp.concatenate([edge_w, jnp.zeros((pad + K,), jnp.float32)]
                          ).reshape(NCH + 1, 1, K)
    zrow = jnp.zeros((K, D), jnp.float32)
    zws = jnp.zeros((2 * RPT,), jnp.float32)
    hf, ws = _sc_call(src2, dst2, ew2, feat, zrow, zws)
    xp = jnp.concatenate([x, jnp.zeros((N_PAD - N, D), jnp.float32)])
    out = _tc_call(xp, hf, ws.reshape(N_PAD // 1024, 2, 1024),
                   W_x, W_f, W_w, b_f.reshape(1, D), weights.reshape(1, D))
    return out[:N]


# speedup vs baseline: 1.6588x; 1.6588x over previous
import jax
import jax.numpy as jnp
from jax import lax
from jax.experimental import pallas as pl
from jax.experimental.pallas import tpu as pltpu
from jax.experimental.pallas import tpu_sc as plsc

N = 10000
D = 128
E = 320000

NC = 2
NS = 16
K = 128
CPT0 = 136
CPT1 = 24
NCH = NS * (CPT0 + CPT1)
E_PAD = NCH * K
N_PAD = 10240
RPT = N_PAD // NS
WSB = 2 * N_PAD


def _sc_body(src_hbm, dst_hbm, ew_hbm, feat_hbm, zrow_hbm, zws_hbm,
             hf_out, ws_out,
             sb0, sb1, db0, db1, wb0, wb1, rows0, rows1, wv, di2,
             hf_sh, ws_sh, semg0, semg1, seme0, seme1):
    cid = lax.axis_index("c")
    sid = lax.axis_index("s")
    base = jnp.where(cid == 0, sid * CPT0, NS * CPT0 + sid * CPT1)
    half_chunks = jnp.where(cid == 0, CPT0 // 2, CPT1 // 2)

    pltpu.sync_copy(zrow_hbm, rows0)
    for k in range(RPT // K):
        pltpu.sync_copy(rows0, hf_sh.at[pl.ds(sid * RPT + k * K, K)])
    pltpu.sync_copy(zws_hbm, ws_sh.at[pl.ds(sid * 2 * RPT, 2 * RPT)])

    plsc.subcore_barrier()

    pltpu.async_copy(src_hbm.at[base], sb0, seme0)
    pltpu.async_copy(dst_hbm.at[base], db0, seme0)
    pltpu.async_copy(ew_hbm.at[base], wb0, seme0)

    def chunk(j, sb, db, wb, rows_b, semg, sb_n, db_n, wb_n, seme_n, seme_b):
        pltpu.make_async_copy(src_hbm.at[base], sb, seme_b).wait()
        pltpu.make_async_copy(dst_hbm.at[base], db, seme_b).wait()
        pltpu.make_async_copy(ew_hbm.at[base], wb, seme_b).wait()
        cp = pltpu.async_copy(feat_hbm.at[sb.at[0]], rows_b, semg)
        pltpu.async_copy(src_hbm.at[base + j + 1], sb_n, seme_n)
        pltpu.async_copy(dst_hbm.at[base + j + 1], db_n, seme_n)
        pltpu.async_copy(ew_hbm.at[base + j + 1], wb_n, seme_n)
        for v in range(K // 16):
            w = wb[0, pl.ds(v * 16, 16)]
            d = db[0, pl.ds(v * 16, 16)]
            fp = ((d >> 10) << 11) + (d & 1023)
            wv[0, pl.ds(v * 16, 16)] = jnp.maximum(w, 0.0)
            wv[1, pl.ds(v * 16, 16)] = jnp.maximum(-w, 0.0)
            di2[0, pl.ds(v * 16, 16)] = fp
            di2[1, pl.ds(v * 16, 16)] = fp + 1024
        pltpu.sync_copy(wv.at[0], ws_sh.at[di2.at[0]], add=True)
        pltpu.sync_copy(wv.at[1], ws_sh.at[di2.at[1]], add=True)
        cp.wait()
        pltpu.sync_copy(rows_b, hf_sh.at[db.at[0]], add=True)

    def body(i, carry):
        chunk(2 * i, sb0, db0, wb0, rows0, semg0, sb1, db1, wb1, seme1, seme0)
        chunk(2 * i + 1, sb1, db1, wb1, rows1, semg1, sb0, db0, wb0, seme0,
              seme1)
        return carry

    lax.fori_loop(0, half_chunks, body, 0)
    pltpu.make_async_copy(src_hbm.at[base], sb0, seme0).wait()
    pltpu.make_async_copy(dst_hbm.at[base], db0, seme0).wait()
    pltpu.make_async_copy(ew_hbm.at[base], wb0, seme0).wait()
    plsc.subcore_barrier()

    for k in range(RPT // K):
        pltpu.sync_copy(hf_sh.at[pl.ds(sid * RPT + k * K, K)],
                        hf_out.at[cid, pl.ds(sid * RPT + k * K, K)])
    pltpu.sync_copy(ws_sh.at[pl.ds(sid * 2 * RPT, 2 * RPT)],
                    ws_out.at[cid, pl.ds(sid * 2 * RPT, 2 * RPT)])


def _sc_call(src2, dst2, ew2, feat, zrow, zws):
    mesh = plsc.VectorSubcoreMesh(core_axis_name="c", subcore_axis_name="s")
    f = pl.kernel(
        _sc_body,
        out_type=[
            jax.ShapeDtypeStruct((NC, N_PAD, D), jnp.float32),
            jax.ShapeDtypeStruct((NC, WSB), jnp.float32),
        ],
        mesh=mesh,
        scratch_types=[
            pltpu.VMEM((1, K), jnp.int32),
            pltpu.VMEM((1, K), jnp.int32),
            pltpu.VMEM((1, K), jnp.int32),
            pltpu.VMEM((1, K), jnp.int32),
            pltpu.VMEM((1, K), jnp.float32),
            pltpu.VMEM((1, K), jnp.float32),
            pltpu.VMEM((K, D), jnp.float32),
            pltpu.VMEM((K, D), jnp.float32),
            pltpu.VMEM((2, K), jnp.float32),
            pltpu.VMEM((2, K), jnp.int32),
            pltpu.VMEM_SHARED((N_PAD, D), jnp.float32),
            pltpu.VMEM_SHARED((WSB,), jnp.float32),
            pltpu.SemaphoreType.DMA,
            pltpu.SemaphoreType.DMA,
            pltpu.SemaphoreType.DMA,
            pltpu.SemaphoreType.DMA,
        ],
    )
    return f(src2, dst2, ew2, feat, zrow, zws)


def _tc_epilogue(x_ref, hf_ref, ws_ref, wx_ref, wf_ref, ww_ref, b_ref, wt_ref,
                 out_ref):
    f32 = jnp.float32
    wt = wt_ref[...]
    rw = jnp.concatenate([jnp.maximum(wt, 0.0), jnp.maximum(-wt, 0.0)], axis=0)
    v = lax.dot_general(rw, ww_ref[...], (((1,), (1,)), ((), ())),
                        preferred_element_type=f32)
    s2 = ws_ref[0, 0] + ws_ref[1, 0]
    hf = hf_ref[0] + hf_ref[1]
    acc = lax.dot_general(x_ref[...], wx_ref[...], (((1,), (1,)), ((), ())),
                          preferred_element_type=f32)
    acc += lax.dot_general(hf, wf_ref[...], (((1,), (1,)), ((), ())),
                           preferred_element_type=f32)
    acc += lax.dot_general(s2, v, (((0,), (0,)), ((), ())),
                           preferred_element_type=f32)
    acc += b_ref[...]
    out_ref[...] = jnp.maximum(acc, 0.0)


def _tc_call(x, hf, ws, W_x, W_f, W_w, b_f, weights):
    blk = 1024
    grid = (N_PAD // blk,)
    return pl.pallas_call(
        _tc_epilogue,
        grid=grid,
        in_specs=[
            pl.BlockSpec((blk, D), lambda i: (i, 0)),
            pl.BlockSpec((NC, blk, D), lambda i: (0, i, 0)),
            pl.BlockSpec((NC, 1, 2, blk), lambda i: (0, i, 0, 0)),
            pl.BlockSpec((D, D), lambda i: (0, 0)),
            pl.BlockSpec((D, D), lambda i: (0, 0)),
            pl.BlockSpec((D, D), lambda i: (0, 0)),
            pl.BlockSpec((1, D), lambda i: (0, 0)),
            pl.BlockSpec((1, D), lambda i: (0, 0)),
        ],
        out_specs=pl.BlockSpec((blk, D), lambda i: (i, 0)),
        out_shape=jax.ShapeDtypeStruct((N_PAD, D), jnp.float32),
    )(x, hf, ws, W_x, W_f, W_w, b_f, weights)


@jax.jit
def kernel(x, feat, edge_index, edge_w, W_x, W_w, W_f, b_f, weights):
    src = edge_index[0].astype(jnp.int32)
    dst = edge_index[1].astype(jnp.int32)
    pad = E_PAD - E
    pad_dst = N + jnp.arange(pad, dtype=jnp.int32) % (N_PAD - N)
    zk = jnp.zeros((K,), jnp.int32)
    src2 = jnp.concatenate([src, jnp.zeros((pad,), jnp.int32),
                            zk]).reshape(NCH + 1, 1, K)
    dst2 = jnp.concatenate([dst, pad_dst,
                            zk + N]).reshape(NCH + 1, 1, K)
    ew2 = jnp.concatenate([edge_w, jnp.zeros((pad + K,), jnp.float32)]
                          ).reshape(NCH + 1, 1, K)
    zrow = jnp.zeros((K, D), jnp.float32)
    zws = jnp.zeros((2 * RPT,), jnp.float32)
    hf, ws = _sc_call(src2, dst2, ew2, feat, zrow, zws)
    xp = jnp.concatenate([x, jnp.zeros((N_PAD - N, D), jnp.float32)])
    out = _tc_call(xp, hf, ws.reshape(NC, N_PAD // 1024, 2, 1024),
                   W_x, W_f, W_w, b_f.reshape(1, D), weights.reshape(1, D))
    return out[:N]
